# Initial kernel scaffold; baseline (speedup 1.0000x reference)
import jax
import jax.numpy as jnp
from jax.experimental import pallas as pl


def kernel(x, params, edge_index, batch):
    raise NotImplementedError("write your pallas kernel here")


# speedup vs baseline: 21.1449x; 21.1449x over previous
import functools

import jax
import jax.numpy as jnp
from jax import lax
from jax.experimental import pallas as pl
from jax.experimental.pallas import tpu as pltpu
from jax.experimental.pallas import tpu_sc as plsc

N = 10000
E = 320000
D = 128
NUM_GRAPHS = 64
OUT = 112
EPS = 1e-7

NS = 16
NC = 2
EPT = E // NS
GW = 128
NGRP = EPT // GW
TAIL = EPT - NGRP * GW
NP = 10240
ROWS_PER_TILE = NP // NS
R = 2000
GRID = N // R

_HI = jax.lax.Precision.DEFAULT


def _dot(a, b):
    return jax.lax.dot_general(a, b, (((1,), (0,)), ((), ())),
                               precision=_HI, preferred_element_type=jnp.float32)


def _ln_in(h, w, b):
    mu = jnp.mean(h, axis=-1, keepdims=True)
    var = jnp.mean((h - mu) ** 2, axis=-1, keepdims=True)
    return (h - mu) / jnp.sqrt(var + 1e-5) * w + b


def _enc_body(x_ref, w_ref, b_ref, t_ref, h0_ref, tab_ref):
    h0 = _dot(x_ref[...], w_ref[...]) + b_ref[...]
    h0_ref[...] = h0
    u = jax.nn.relu(h0) + EPS
    a = jnp.exp(u * t_ref[0, 0])
    tab_ref[0] = a
    tab_ref[1] = u * a


def _mid_body(dn_ref, nm_ref, h0_ref, w1_ref, b1_ref, lw_ref, lb_ref,
              w2_ref, b2_ref, l1w_ref, l1b_ref, t_ref,
              h1_ref, tp_ref, tab_ref):
    aggr = nm_ref[...] / (dn_ref[...] + 1e-16)
    out = aggr + h0_ref[...]
    h = _dot(out, w1_ref[...]) + b1_ref[...]
    h = _ln_in(h, lw_ref[...], lb_ref[...])
    h = jax.nn.relu(h)
    h1 = _dot(h, w2_ref[...]) + b2_ref[...]
    h1_ref[...] = h1
    tp = jax.nn.relu(_ln_in(h1, l1w_ref[...], l1b_ref[...]))
    tp_ref[...] = tp
    u = tp + EPS
    a = jnp.exp(u * t_ref[0, 0])
    tab_ref[0] = a
    tab_ref[1] = u * a


def _fin_body(dn_ref, nm_ref, tp_ref, h1_ref, w1_ref, b1_ref, lw_ref, lb_ref,
              w2_ref, b2_ref, l0w_ref, l0b_ref, bt_ref,
              wm1_ref, bm1_ref, wm2_ref, bm2_ref,
              out_ref, pool_acc):
    j = pl.program_id(0)
    aggr = nm_ref[...] / (dn_ref[...] + 1e-16)
    out = aggr + tp_ref[...]
    h = _dot(out, w1_ref[...]) + b1_ref[...]
    h = _ln_in(h, lw_ref[...], lb_ref[...])
    h = jax.nn.relu(h)
    h2 = h1_ref[...] + _dot(h, w2_ref[...]) + b2_ref[...]
    h3 = jax.nn.relu(_ln_in(h2, l0w_ref[...], l0b_ref[...]))
    seg = bt_ref[0]
    mask = (jax.lax.broadcasted_iota(jnp.int32, (NUM_GRAPHS, R), 0)
            == seg).astype(jnp.float32)
    contrib = _dot(mask, h3)

    @pl.when(j == 0)
    def _():
        pool_acc[...] = jnp.zeros_like(pool_acc)

    pool_acc[...] += contrib

    @pl.when(j == GRID - 1)
    def _():
        p = pool_acc[...]
        o = jax.nn.relu(_dot(p, wm1_ref[...]) + bm1_ref[...])
        out_ref[...] = _dot(o, wm2_ref[...]) + bm2_ref[...]


def _sc_body(tab_hbm, srcs_hbm, dst_hbm, dn_hbm, nm_hbm,
             sl0, sl1, sl2, dl0, dl1, dl2, rows0, rows1,
             sidxt, didxt, acc, isem, gsem, ssem):
    cid = lax.axis_index("c")
    tid = lax.axis_index("s")
    sl = (sl0, sl1, sl2)
    dl = (dl0, dl1, dl2)
    rows = (rows0, rows1)
    ebase = tid * EPT

    def _idx_issue(g, q):
        off = ebase + g * GW
        pltpu.async_copy(srcs_hbm.at[pl.ds(cid * E + off, GW)], sl[q], isem)
        pltpu.async_copy(dst_hbm.at[pl.ds(off, GW)], dl[q], isem)

    def _idx_wait(q):
        pltpu.make_async_copy(srcs_hbm.at[pl.ds(0, GW)], sl[q], isem).wait()
        pltpu.make_async_copy(dst_hbm.at[pl.ds(0, GW)], dl[q], isem).wait()

    def _gather_issue(q, b):
        pltpu.async_copy(tab_hbm.at[sl[q]], rows[b], gsem)

    def _gather_wait(q, b):
        pltpu.make_async_copy(tab_hbm.at[sl[q]], rows[b], gsem).wait()

    def _scatter_issue(q, b):
        pltpu.async_copy(rows[b], acc.at[dl[q]], ssem, add=True)

    def _scatter_wait(q, b):
        pltpu.make_async_copy(rows[b], acc.at[dl[q]], ssem).wait()

    _idx_issue(0, 0)
    _idx_issue(1, 1)

    def _zrow(i, _):
        for k in range(8):
            rows0[i, pl.ds(16 * k, 16)] = jnp.zeros((16,), jnp.float32)
        return 0
    lax.fori_loop(0, GW, _zrow, 0)
    for q in range(ROWS_PER_TILE // GW):
        pltpu.sync_copy(rows0, acc.at[pl.ds(tid * ROWS_PER_TILE + q * GW, GW)])
    plsc.subcore_barrier()
    _idx_wait(0)
    _gather_issue(0, 0)

    def _six(i, _):
        for u in range(6):
            g = 6 * i + u
            q, b = u % 3, u % 2
            qn, bn = (u + 1) % 3, 1 - b
            qp = (u - 1) % 3
            pl.when(g + 1 < NGRP)(lambda q=qn: _idx_wait(q))
            if u == 0:
                pl.when(g > 0)(lambda: _scatter_wait(qp, bn))
            else:
                _scatter_wait(qp, bn)
            pl.when(g + 1 < NGRP)(lambda q=qn, b=bn: _gather_issue(q, b))
            _gather_wait(q, b)
            _scatter_issue(q, b)
            pl.when(g + 2 < NGRP)(
                lambda g=g, q=(u + 2) % 3: _idx_issue(g + 2, q))
        return 0
    lax.fori_loop(0, NGRP // 6, _six, 0)

    _scatter_wait((NGRP - 1) % 3, (NGRP - 1) % 2)
    tb = ebase + NGRP * GW
    pltpu.sync_copy(srcs_hbm.at[pl.ds(cid * E + tb, TAIL)], sidxt)
    pltpu.sync_copy(dst_hbm.at[pl.ds(tb, TAIL)], didxt)
    tr = rows0.at[pl.ds(0, TAIL)]
    pltpu.async_copy(tab_hbm.at[sidxt], tr, gsem).wait()
    pltpu.sync_copy(tr, acc.at[didxt], add=True)

    plsc.subcore_barrier()
    rbase = tid * ROWS_PER_TILE

    @pl.when(cid == 0)
    def _():
        pltpu.sync_copy(acc.at[pl.ds(rbase, ROWS_PER_TILE)],
                        dn_hbm.at[pl.ds(rbase, ROWS_PER_TILE)])

    @pl.when(cid == 1)
    def _():
        pltpu.sync_copy(acc.at[pl.ds(rbase, ROWS_PER_TILE)],
                        nm_hbm.at[pl.ds(rbase, ROWS_PER_TILE)])


@functools.cache
def _sc_segsum():
    return pl.kernel(
        _sc_body,
        out_type=(jax.ShapeDtypeStruct((NP, D), jnp.float32),
                  jax.ShapeDtypeStruct((NP, D), jnp.float32)),
        mesh=plsc.VectorSubcoreMesh(core_axis_name="c", subcore_axis_name="s",
                                    num_cores=NC, num_subcores=NS),
        scratch_types=(
            [pltpu.VMEM((GW,), jnp.int32)] * 6
            + [pltpu.VMEM((GW, D), jnp.float32)] * 2
            + [pltpu.VMEM((TAIL,), jnp.int32)] * 2
            + [pltpu.VMEM_SHARED((NP, D), jnp.float32)]
            + [pltpu.SemaphoreType.DMA] * 3
        ),
    )


def _row_spec(offset_blocks=0):
    return pl.BlockSpec((R, D), lambda j, o=offset_blocks: (j + o, 0))


def _full(shape):
    return pl.BlockSpec(shape, lambda j: tuple(0 for _ in shape))


def kernel(x, params, edge_index, batch):
    p = params
    src = edge_index[0].astype(jnp.int32)
    dst = edge_index[1].astype(jnp.int32)
    srcs = jnp.concatenate([src, src + N])
    batch3 = batch.astype(jnp.int32).reshape(GRID, 1, R)

    def v2(a):
        return a.reshape(1, -1)

    t1 = p['conv1']['t'].reshape(1, 1)
    t2 = p['conv2']['t'].reshape(1, 1)

    h0, tab1 = pl.pallas_call(
        _enc_body,
        grid=(GRID,),
        in_specs=[_row_spec(), _full((D, D)), _full((1, D)), _full((1, 1))],
        out_specs=[_row_spec(),
                   pl.BlockSpec((2, R, D), lambda j: (0, j, 0))],
        out_shape=[jax.ShapeDtypeStruct((N, D), jnp.float32),
                   jax.ShapeDtypeStruct((2, N, D), jnp.float32)],
    )(x, p['W_enc'], v2(p['b_enc']), t1)

    dn1, nm1 = _sc_segsum()(tab1.reshape(2 * N, D), srcs, dst)

    c1 = p['conv1']
    h1, tpre, tab2 = pl.pallas_call(
        _mid_body,
        grid=(GRID,),
        in_specs=[_row_spec(), _row_spec(), _row_spec(),
                  _full((D, 2 * D)), _full((1, 2 * D)), _full((1, 2 * D)),
                  _full((1, 2 * D)), _full((2 * D, D)), _full((1, D)),
                  _full((1, D)), _full((1, D)), _full((1, 1))],
        out_specs=[_row_spec(), _row_spec(),
                   pl.BlockSpec((2, R, D), lambda j: (0, j, 0))],
        out_shape=[jax.ShapeDtypeStruct((N, D), jnp.float32),
                   jax.ShapeDtypeStruct((N, D), jnp.float32),
                   jax.ShapeDtypeStruct((2, N, D), jnp.float32)],
    )(dn1, nm1, h0, c1['W1'], v2(c1['b1']), v2(c1['ln_w']), v2(c1['ln_b']),
      c1['W2'], v2(c1['b2']), v2(p['ln1_w']), v2(p['ln1_b']), t2)

    dn2, nm2 = _sc_segsum()(tab2.reshape(2 * N, D), srcs, dst)

    c2 = p['conv2']
    out = pl.pallas_call(
        _fin_body,
        grid=(GRID,),
        in_specs=[_row_spec(), _row_spec(), _row_spec(), _row_spec(),
                  _full((D, 2 * D)), _full((1, 2 * D)), _full((1, 2 * D)),
                  _full((1, 2 * D)), _full((2 * D, D)), _full((1, D)),
                  _full((1, D)), _full((1, D)),
                  pl.BlockSpec((1, 1, R), lambda j: (j, 0, 0)),
                  _full((D, D // 2)), _full((1, D // 2)),
                  _full((D // 2, OUT)), _full((1, OUT))],
        out_specs=pl.BlockSpec((NUM_GRAPHS, OUT), lambda j: (0, 0)),
        out_shape=jax.ShapeDtypeStruct((NUM_GRAPHS, OUT), jnp.float32),
        scratch_shapes=[pltpu.VMEM((NUM_GRAPHS, D), jnp.float32)],
    )(dn2, nm2, tpre, h1, c2['W1'], v2(c2['b1']), v2(c2['ln_w']),
      v2(c2['ln_b']), c2['W2'], v2(c2['b2']), v2(p['ln0_w']), v2(p['ln0_b']),
      batch3, p['Wm1'], v2(p['bm1']), p['Wm2'], v2(p['bm2']))

    return out
